# baseline (speedup 1.0000x reference)
import functools

import jax
import jax.numpy as jnp
from jax import lax
from jax.experimental import pallas as pl
from jax.experimental.pallas import tpu as pltpu
from jax.experimental.pallas import tpu_sc as plsc

NC = 2
NS = 16
CH = 128
CHA = 56


def _sc_mesh():
  return plsc.VectorSubcoreMesh(
      core_axis_name="c", subcore_axis_name="s",
      num_cores=NC, num_subcores=NS)


def _deg_partials(dst4, zinit, ones, n_pad, n_chunks):
  rows_per_tile = n_pad // NS

  @functools.partial(
      pl.kernel,
      out_type=jax.ShapeDtypeStruct((NC, n_pad, 16), jnp.float32),
      mesh=_sc_mesh(),
      compiler_params=pltpu.CompilerParams(use_tc_tiling_on_sc=False),
      scratch_types=[
          pltpu.VMEM((CH,), jnp.int32),
          pltpu.VMEM((CH, 16), jnp.float32),
          pltpu.VMEM_SHARED((n_pad, 16), jnp.float32),
      ],
  )
  def run(dst_hbm, z_hbm, ones_hbm, out_hbm, idx_v, ones_v, acc_sh):
    c = lax.axis_index("c")
    s = lax.axis_index("s")
    r0 = s * rows_per_tile
    pltpu.sync_copy(z_hbm.at[pl.ds(r0, rows_per_tile)],
                    acc_sh.at[pl.ds(r0, rows_per_tile)])
    pltpu.sync_copy(ones_hbm, ones_v)
    plsc.subcore_barrier()

    def body(ch, carry):
      pltpu.sync_copy(dst_hbm.at[c, s, ch], idx_v)
      pltpu.sync_copy(ones_v, acc_sh.at[idx_v], add=True)
      return carry

    lax.fori_loop(0, n_chunks, body, 0)
    plsc.subcore_barrier()
    pltpu.sync_copy(acc_sh.at[pl.ds(r0, rows_per_tile)],
                    out_hbm.at[c, pl.ds(r0, rows_per_tile)])

  return run(dst4, zinit, ones)


def _gcn_aggregate(xw, srcs, dsts, n_nodes, n_pad, n_chunks, n_graphs):
  rows_per_tile = n_nodes // NS
  graphs_per_core = n_graphs // NC
  K = 4
  assert n_chunks % K == 0

  @functools.partial(
      pl.kernel,
      out_type=jax.ShapeDtypeStruct(xw.shape, jnp.float32),
      mesh=_sc_mesh(),
      compiler_params=pltpu.CompilerParams(use_tc_tiling_on_sc=False),
      scratch_types=[
          pltpu.VMEM((n_chunks, CHA), jnp.int32),
          pltpu.VMEM((n_chunks, CHA), jnp.int32),
          pltpu.VMEM((CHA, 128), jnp.float32),
          pltpu.VMEM((CHA, 128), jnp.float32),
          pltpu.VMEM((CHA, 128), jnp.float32),
          pltpu.VMEM((CHA, 128), jnp.float32),
          pltpu.VMEM_SHARED((n_pad, 128), jnp.float32),
          pltpu.SemaphoreType.DMA,
          pltpu.SemaphoreType.DMA,
          pltpu.SemaphoreType.DMA,
          pltpu.SemaphoreType.DMA,
          pltpu.SemaphoreType.DMA,
          pltpu.SemaphoreType.DMA,
          pltpu.SemaphoreType.DMA,
          pltpu.SemaphoreType.DMA,
      ],
  )
  def run(xw_hbm, srcs_hbm, dsts_hbm, out_hbm, sidx_v, didx_v,
          b0, b1, b2, b3, acc_sh, g0, g1, g2, g3, s0, s1, s2, s3):
    c = lax.axis_index("c")
    s = lax.axis_index("s")
    r0 = s * rows_per_tile
    bufs = [b0, b1, b2, b3]
    gsems = [g0, g1, g2, g3]
    ssems = [s0, s1, s2, s3]

    def drain(b, sem_list):
      pltpu.make_async_copy(xw_hbm.at[pl.ds(0, CHA)], bufs[b],
                            sem_list[b]).wait()

    pltpu.sync_copy(dsts_hbm.at[s], didx_v)
    for gg in range(graphs_per_core):
      g = c * graphs_per_core + gg
      base = g * n_nodes
      pltpu.sync_copy(xw_hbm.at[pl.ds(base + r0, rows_per_tile)],
                      acc_sh.at[pl.ds(r0, rows_per_tile)])
      pltpu.sync_copy(srcs_hbm.at[g, s], sidx_v)
      plsc.subcore_barrier()

      pltpu.async_copy(xw_hbm.at[sidx_v.at[0]], bufs[0], gsems[0])
      pltpu.async_copy(xw_hbm.at[sidx_v.at[1]], bufs[1], gsems[1])

      def body(j, carry):
        for b in range(K):
          v = K * j + b
          drain(b, gsems)
          pltpu.async_copy(bufs[b], acc_sh.at[didx_v.at[v]],
                           ssems[b], add=True)

          @pl.when(v >= 2)
          def _(b2=(b - 2) % K):
            drain(b2, ssems)

          @pl.when(v + 2 < n_chunks)
          def _(b2=(b + 2) % K, v=v):
            pltpu.async_copy(xw_hbm.at[sidx_v.at[v + 2]], bufs[b2],
                             gsems[b2])

        return carry

      lax.fori_loop(0, n_chunks // K, body, 0)
      drain((n_chunks - 2) % K, ssems)
      drain((n_chunks - 1) % K, ssems)
      plsc.subcore_barrier()
      pltpu.sync_copy(acc_sh.at[pl.ds(r0, rows_per_tile)],
                      out_hbm.at[pl.ds(base + r0, rows_per_tile)])
      plsc.subcore_barrier()

  return run(xw, srcs, dsts)


def _silu(x):
  return x * jax.nn.sigmoid(x)


def _dinv_of(p):
  deg = 1.0 + p[0, :, 0:1] + p[1, :, 0:1]
  return lax.rsqrt(deg)


def _pre_matmul(xs, scale, shift, w, p, blk, reps):
  nblk = xs.shape[0] // blk
  d = w.shape[0]

  def body(xs_ref, sc_ref, sh_ref, w_ref, p_ref, out_ref):
    dinv = _dinv_of(p_ref[...])
    x = xs_ref[...] * sc_ref[...] + sh_ref[...]
    y = lax.dot_general(x, w_ref[...], (((1,), (1,)), ((), ())),
                        preferred_element_type=jnp.float32)
    out_ref[...] = y * dinv

  return pl.pallas_call(
      body,
      grid=(nblk,),
      in_specs=[
          pl.BlockSpec((blk, xs.shape[1]), lambda b: (b, 0)),
          pl.BlockSpec((1, xs.shape[1]), lambda b: (0, 0)),
          pl.BlockSpec((1, xs.shape[1]), lambda b: (0, 0)),
          pl.BlockSpec(w.shape, lambda b: (0, 0)),
          pl.BlockSpec((NC, blk, 16), lambda b: (0, b % reps, 0)),
      ],
      out_specs=pl.BlockSpec((blk, d), lambda b: (b, 0)),
      out_shape=jax.ShapeDtypeStruct((xs.shape[0], d), jnp.float32),
  )(xs, scale, shift, w, p)


def _mid_matmul(s, b, w, p, blk, reps):
  nblk = s.shape[0] // blk
  d = w.shape[0]

  def body(s_ref, b_ref, w_ref, p_ref, out_ref):
    dinv = _dinv_of(p_ref[...])
    h = _silu(s_ref[...] * dinv + b_ref[...])
    y = lax.dot_general(h, w_ref[...], (((1,), (1,)), ((), ())),
                        preferred_element_type=jnp.float32)
    out_ref[...] = y * dinv

  return pl.pallas_call(
      body,
      grid=(nblk,),
      in_specs=[
          pl.BlockSpec((blk, s.shape[1]), lambda b: (b, 0)),
          pl.BlockSpec((1, d), lambda b: (0, 0)),
          pl.BlockSpec(w.shape, lambda b: (0, 0)),
          pl.BlockSpec((NC, blk, 16), lambda b: (0, b % reps, 0)),
      ],
      out_specs=pl.BlockSpec((blk, d), lambda b: (b, 0)),
      out_shape=jax.ShapeDtypeStruct((s.shape[0], d), jnp.float32),
  )(s, b, w, p)


def _pool_sums(s, b, p, blk, reps, n_graphs):
  nblk = s.shape[0] // blk
  d = s.shape[1]

  def body(s_ref, b_ref, p_ref, out_ref):
    j = pl.program_id(0) % reps
    dinv = _dinv_of(p_ref[...])
    h = _silu(s_ref[...] * dinv + b_ref[...])
    part = jnp.sum(h, axis=0, keepdims=True)[None]

    @pl.when(j == 0)
    def _():
      out_ref[...] = part

    @pl.when(j != 0)
    def _():
      out_ref[...] += part

  return pl.pallas_call(
      body,
      grid=(nblk,),
      in_specs=[
          pl.BlockSpec((blk, d), lambda b: (b, 0)),
          pl.BlockSpec((1, d), lambda b: (0, 0)),
          pl.BlockSpec((NC, blk, 16), lambda b: (0, b % reps, 0)),
      ],
      out_specs=pl.BlockSpec((1, 1, d), lambda b: (b // reps, 0, 0)),
      out_shape=jax.ShapeDtypeStruct((n_graphs, 1, d), jnp.float32),
  )(s, b, p)


def _seq_head(pool_sums, n_nodes,
              wih0, whh0, bi0, bh0, wih1, whh1, bi1, bh1,
              wh1, bhd1, wh2, bhd2, wh3, bhd3):
  t_steps, d = pool_sums.shape

  def dotT(x, w):
    return lax.dot_general(x, w, (((1,), (1,)), ((), ())),
                           preferred_element_type=jnp.float32)

  def body(pool_ref, wih0_r, whh0_r, bi0_r, bh0_r,
           wih1_r, whh1_r, bi1_r, bh1_r,
           wh1_r, bhd1_r, wh2_r, bhd2_r, wh3_r, bhd3_r, out_ref):
    emb = pool_ref[...] * (1.0 / n_nodes)

    def cell(xt, h, c, wih, whh, bi, bh):
      gates = dotT(xt, wih) + dotT(h, whh) + bi + bh
      i = jax.nn.sigmoid(gates[:, 0 * d:1 * d])
      f = jax.nn.sigmoid(gates[:, 1 * d:2 * d])
      g = jnp.tanh(gates[:, 2 * d:3 * d])
      o = jax.nn.sigmoid(gates[:, 3 * d:4 * d])
      c = f * c + i * g
      h = o * jnp.tanh(c)
      return h, c

    h0 = jnp.zeros((1, d), jnp.float32)
    c0 = jnp.zeros((1, d), jnp.float32)
    ys = []
    for t in range(t_steps):
      h0, c0 = cell(emb[t:t + 1, :], h0, c0,
                    wih0_r[...], whh0_r[...], bi0_r[...], bh0_r[...])
      ys.append(h0)
    h1 = jnp.zeros((1, d), jnp.float32)
    c1 = jnp.zeros((1, d), jnp.float32)
    for t in range(t_steps):
      h1, c1 = cell(ys[t], h1, c1,
                    wih1_r[...], whh1_r[...], bi1_r[...], bh1_r[...])
    hh = _silu(dotT(h1, wh1_r[...]) + bhd1_r[...])
    hh = _silu(dotT(hh, wh2_r[...]) + bhd2_r[...])
    z = dotT(hh, wh3_r[...]) + bhd3_r[...]
    sp = jnp.maximum(z, 0.0) + jnp.log(1.0 + jnp.exp(-jnp.abs(z))) + 1e-6
    out_ref[...] = sp

  return pl.pallas_call(
      body,
      out_shape=jax.ShapeDtypeStruct((1, wh3.shape[0]), jnp.float32),
  )(pool_sums, wih0, whh0, bi0, bh0, wih1, whh1, bi1, bh1,
    wh1, bhd1, wh2, bhd2, wh3, bhd3)


def kernel(snapshot_sequence, edge_index, scale, shift,
           W1, b1, W2, b2, W3, b3,
           Wih0, Whh0, bih0, bhh0, Wih1, Whh1, bih1, bhh1,
           Wh1, bh1, Wh2, bh2, Wh3, bh3):
  B, T, N, F = snapshot_sequence.shape
  D = W1.shape[0]
  G = B * T
  GN = G * N
  E = edge_index.shape[1]

  ei = edge_index.astype(jnp.int32)
  src, dst = ei[0], ei[1]

  n_ch16 = -(-E // (NS * CHA))
  n_ch16 = -(-n_ch16 // 4) * 4
  pad16 = NS * CHA * n_ch16 - E
  src_p = jnp.concatenate([src, jnp.zeros((pad16,), jnp.int32)])
  dst_p = jnp.concatenate([dst, jnp.full((pad16,), N, jnp.int32)])
  offs = (jnp.arange(G, dtype=jnp.int32) * N)[:, None]
  srcs4 = (src_p[None, :] + offs).reshape(G, NS, n_ch16, CHA)
  dsts3 = dst_p.reshape(NS, n_ch16, CHA)

  n_ch32 = -(-E // (NC * NS * CH))
  pad32 = NC * NS * CH * n_ch32 - E
  dst_deg = jnp.concatenate(
      [dst, jnp.full((pad32,), N, jnp.int32)]).reshape(NC, NS, n_ch32, CH)

  n_pad = (N // NS + 1) * NS
  n_pad_a = N + 8
  zinit = jnp.zeros((n_pad, 16), jnp.float32)
  ones = jnp.ones((CH, 16), jnp.float32)

  p = _deg_partials(dst_deg, zinit, ones, n_pad, n_ch32)[:, :N, :]

  blk = 2000
  reps = N // blk

  xs = snapshot_sequence.reshape(GN, F)
  xw = _pre_matmul(xs, scale.reshape(1, F), shift.reshape(1, F), W1, p,
                   blk, reps)
  s1 = _gcn_aggregate(xw, srcs4, dsts3, N, n_pad_a, n_ch16, G)
  xw2 = _mid_matmul(s1, b1.reshape(1, D), W2, p, blk, reps)
  s2 = _gcn_aggregate(xw2, srcs4, dsts3, N, n_pad_a, n_ch16, G)
  xw3 = _mid_matmul(s2, b2.reshape(1, D), W3, p, blk, reps)
  s3 = _gcn_aggregate(xw3, srcs4, dsts3, N, n_pad_a, n_ch16, G)

  ps = _pool_sums(s3, b3.reshape(1, D), p, blk, reps, G)
  n_out = Wh3.shape[0]
  wh3p = jnp.concatenate(
      [Wh3, jnp.zeros((128 - n_out, Wh3.shape[1]), jnp.float32)], axis=0)
  bh3p = jnp.concatenate(
      [bh3, jnp.zeros((128 - n_out,), jnp.float32)]).reshape(1, 128)
  res = _seq_head(ps.reshape(G, D), N,
                  Wih0, Whh0, bih0.reshape(1, 4 * D), bhh0.reshape(1, 4 * D),
                  Wih1, Whh1, bih1.reshape(1, 4 * D), bhh1.reshape(1, 4 * D),
                  Wh1, bh1.reshape(1, -1), Wh2, bh2.reshape(1, -1),
                  wh3p, bh3p)
  return res[:B, :n_out]


# speedup vs baseline: 1.1192x; 1.1192x over previous
import functools

import jax
import jax.numpy as jnp
from jax import lax
from jax.experimental import pallas as pl
from jax.experimental.pallas import tpu as pltpu
from jax.experimental.pallas import tpu_sc as plsc

NC = 2
NS = 16
CH = 128
CHA = 80


def _sc_mesh():
  return plsc.VectorSubcoreMesh(
      core_axis_name="c", subcore_axis_name="s",
      num_cores=NC, num_subcores=NS)


def _deg_partials(dst4, zinit, ones, n_pad, n_chunks):
  rows_per_tile = n_pad // NS

  @functools.partial(
      pl.kernel,
      out_type=jax.ShapeDtypeStruct((NC, n_pad, 16), jnp.float32),
      mesh=_sc_mesh(),
      compiler_params=pltpu.CompilerParams(use_tc_tiling_on_sc=False),
      scratch_types=[
          pltpu.VMEM((CH,), jnp.int32),
          pltpu.VMEM((CH, 16), jnp.float32),
          pltpu.VMEM_SHARED((n_pad, 16), jnp.float32),
      ],
  )
  def run(dst_hbm, z_hbm, ones_hbm, out_hbm, idx_v, ones_v, acc_sh):
    c = lax.axis_index("c")
    s = lax.axis_index("s")
    r0 = s * rows_per_tile
    pltpu.sync_copy(z_hbm.at[pl.ds(r0, rows_per_tile)],
                    acc_sh.at[pl.ds(r0, rows_per_tile)])
    pltpu.sync_copy(ones_hbm, ones_v)
    plsc.subcore_barrier()

    def body(ch, carry):
      pltpu.sync_copy(dst_hbm.at[c, s, ch], idx_v)
      pltpu.sync_copy(ones_v, acc_sh.at[idx_v], add=True)
      return carry

    lax.fori_loop(0, n_chunks, body, 0)
    plsc.subcore_barrier()
    pltpu.sync_copy(acc_sh.at[pl.ds(r0, rows_per_tile)],
                    out_hbm.at[c, pl.ds(r0, rows_per_tile)])

  return run(dst4, zinit, ones)


def _gcn_aggregate(xw, srcs, dsts, n_nodes, n_pad, n_chunks, n_graphs):
  del n_graphs
  rows_per_tile = n_nodes // NS
  K = 3
  assert n_chunks % K == 0

  @functools.partial(
      pl.kernel,
      out_type=jax.ShapeDtypeStruct(xw.shape, jnp.float32),
      mesh=_sc_mesh(),
      compiler_params=pltpu.CompilerParams(use_tc_tiling_on_sc=False),
      scratch_types=[
          pltpu.VMEM((n_chunks, CHA), jnp.int32),
          pltpu.VMEM((n_chunks, CHA), jnp.int32),
          pltpu.VMEM((CHA, 128), jnp.float32),
          pltpu.VMEM((CHA, 128), jnp.float32),
          pltpu.VMEM((CHA, 128), jnp.float32),
          pltpu.VMEM_SHARED((n_pad, 128), jnp.float32),
          pltpu.SemaphoreType.DMA,
          pltpu.SemaphoreType.DMA,
          pltpu.SemaphoreType.DMA,
          pltpu.SemaphoreType.DMA,
          pltpu.SemaphoreType.DMA,
          pltpu.SemaphoreType.DMA,
      ],
  )
  def run(xw_hbm, srcs_hbm, dsts_hbm, out_hbm, sidx_v, didx_v,
          b0, b1, b2, acc_sh, g0, g1, g2, s0, s1, s2):
    c = lax.axis_index("c")
    s = lax.axis_index("s")
    r0 = s * rows_per_tile
    bufs = [b0, b1, b2]
    gsems = [g0, g1, g2]
    ssems = [s0, s1, s2]

    def drain(b, sem_list):
      pltpu.make_async_copy(xw_hbm.at[pl.ds(0, CHA)], bufs[b],
                            sem_list[b]).wait()

    pltpu.sync_copy(dsts_hbm.at[s], didx_v)
    base = c * n_nodes
    pltpu.sync_copy(xw_hbm.at[pl.ds(base + r0, rows_per_tile)],
                    acc_sh.at[pl.ds(r0, rows_per_tile)])
    pltpu.sync_copy(srcs_hbm.at[c, s], sidx_v)
    plsc.subcore_barrier()

    pltpu.async_copy(xw_hbm.at[sidx_v.at[0]], bufs[0], gsems[0])
    pltpu.async_copy(xw_hbm.at[sidx_v.at[1]], bufs[1], gsems[1])

    def body(j, carry):
      for b in range(K):
        v = K * j + b
        drain(b, gsems)
        pltpu.async_copy(bufs[b], acc_sh.at[didx_v.at[v]],
                         ssems[b], add=True)

        @pl.when(v >= 1)
        def _(b2=(b - 1) % K):
          drain(b2, ssems)

        @pl.when(v + 2 < n_chunks)
        def _(b2=(b + 2) % K, v=v):
          pltpu.async_copy(xw_hbm.at[sidx_v.at[v + 2]], bufs[b2],
                           gsems[b2])

      return carry

    lax.fori_loop(0, n_chunks // K, body, 0)
    drain((n_chunks - 1) % K, ssems)
    plsc.subcore_barrier()
    pltpu.sync_copy(acc_sh.at[pl.ds(r0, rows_per_tile)],
                    out_hbm.at[pl.ds(base + r0, rows_per_tile)])

  return run(xw, srcs, dsts)


def _silu(x):
  return x * jax.nn.sigmoid(x)


def _dinv_of(p):
  deg = 1.0 + p[0, :, 0:1] + p[1, :, 0:1]
  return lax.rsqrt(deg)


def _pre_matmul(xs, scale, shift, w, p, blk, reps):
  nblk = xs.shape[0] // blk
  d = w.shape[0]

  def body(xs_ref, sc_ref, sh_ref, w_ref, p_ref, out_ref):
    dinv = _dinv_of(p_ref[...])
    x = xs_ref[...] * sc_ref[...] + sh_ref[...]
    y = lax.dot_general(x, w_ref[...], (((1,), (1,)), ((), ())),
                        preferred_element_type=jnp.float32)
    out_ref[...] = y * dinv

  return pl.pallas_call(
      body,
      grid=(nblk,),
      in_specs=[
          pl.BlockSpec((blk, xs.shape[1]), lambda b: (b, 0)),
          pl.BlockSpec((1, xs.shape[1]), lambda b: (0, 0)),
          pl.BlockSpec((1, xs.shape[1]), lambda b: (0, 0)),
          pl.BlockSpec(w.shape, lambda b: (0, 0)),
          pl.BlockSpec((NC, blk, 16), lambda b: (0, b % reps, 0)),
      ],
      out_specs=pl.BlockSpec((blk, d), lambda b: (b, 0)),
      out_shape=jax.ShapeDtypeStruct((xs.shape[0], d), jnp.float32),
  )(xs, scale, shift, w, p)


def _mid_matmul(s, b, w, p, blk, reps):
  nblk = s.shape[0] // blk
  d = w.shape[0]

  def body(s_ref, b_ref, w_ref, p_ref, out_ref):
    dinv = _dinv_of(p_ref[...])
    h = _silu(s_ref[...] * dinv + b_ref[...])
    y = lax.dot_general(h, w_ref[...], (((1,), (1,)), ((), ())),
                        preferred_element_type=jnp.float32)
    out_ref[...] = y * dinv

  return pl.pallas_call(
      body,
      grid=(nblk,),
      in_specs=[
          pl.BlockSpec((blk, s.shape[1]), lambda b: (b, 0)),
          pl.BlockSpec((1, d), lambda b: (0, 0)),
          pl.BlockSpec(w.shape, lambda b: (0, 0)),
          pl.BlockSpec((NC, blk, 16), lambda b: (0, b % reps, 0)),
      ],
      out_specs=pl.BlockSpec((blk, d), lambda b: (b, 0)),
      out_shape=jax.ShapeDtypeStruct((s.shape[0], d), jnp.float32),
  )(s, b, w, p)


def _pool_sums(s, b, p, blk, reps, n_graphs):
  nblk = s.shape[0] // blk
  d = s.shape[1]

  def body(s_ref, b_ref, p_ref, out_ref):
    j = pl.program_id(0) % reps
    dinv = _dinv_of(p_ref[...])
    h = _silu(s_ref[...] * dinv + b_ref[...])
    part = jnp.sum(h, axis=0, keepdims=True)[None]

    @pl.when(j == 0)
    def _():
      out_ref[...] = part

    @pl.when(j != 0)
    def _():
      out_ref[...] += part

  return pl.pallas_call(
      body,
      grid=(nblk,),
      in_specs=[
          pl.BlockSpec((blk, d), lambda b: (b, 0)),
          pl.BlockSpec((1, d), lambda b: (0, 0)),
          pl.BlockSpec((NC, blk, 16), lambda b: (0, b % reps, 0)),
      ],
      out_specs=pl.BlockSpec((1, 1, d), lambda b: (b // reps, 0, 0)),
      out_shape=jax.ShapeDtypeStruct((n_graphs, 1, d), jnp.float32),
  )(s, b, p)


def _seq_head(pool_sums, n_nodes,
              wih0, whh0, bi0, bh0, wih1, whh1, bi1, bh1,
              wh1, bhd1, wh2, bhd2, wh3, bhd3):
  t_steps, d = pool_sums.shape

  def dotT(x, w):
    return lax.dot_general(x, w, (((1,), (1,)), ((), ())),
                           preferred_element_type=jnp.float32)

  def body(pool_ref, wih0_r, whh0_r, bi0_r, bh0_r,
           wih1_r, whh1_r, bi1_r, bh1_r,
           wh1_r, bhd1_r, wh2_r, bhd2_r, wh3_r, bhd3_r, out_ref):
    emb = pool_ref[...] * (1.0 / n_nodes)

    def cell(xt, h, c, wih, whh, bi, bh):
      gates = dotT(xt, wih) + dotT(h, whh) + bi + bh
      i = jax.nn.sigmoid(gates[:, 0 * d:1 * d])
      f = jax.nn.sigmoid(gates[:, 1 * d:2 * d])
      g = jnp.tanh(gates[:, 2 * d:3 * d])
      o = jax.nn.sigmoid(gates[:, 3 * d:4 * d])
      c = f * c + i * g
      h = o * jnp.tanh(c)
      return h, c

    h0 = jnp.zeros((1, d), jnp.float32)
    c0 = jnp.zeros((1, d), jnp.float32)
    ys = []
    for t in range(t_steps):
      h0, c0 = cell(emb[t:t + 1, :], h0, c0,
                    wih0_r[...], whh0_r[...], bi0_r[...], bh0_r[...])
      ys.append(h0)
    h1 = jnp.zeros((1, d), jnp.float32)
    c1 = jnp.zeros((1, d), jnp.float32)
    for t in range(t_steps):
      h1, c1 = cell(ys[t], h1, c1,
                    wih1_r[...], whh1_r[...], bi1_r[...], bh1_r[...])
    hh = _silu(dotT(h1, wh1_r[...]) + bhd1_r[...])
    hh = _silu(dotT(hh, wh2_r[...]) + bhd2_r[...])
    z = dotT(hh, wh3_r[...]) + bhd3_r[...]
    sp = jnp.maximum(z, 0.0) + jnp.log(1.0 + jnp.exp(-jnp.abs(z))) + 1e-6
    out_ref[...] = sp

  return pl.pallas_call(
      body,
      out_shape=jax.ShapeDtypeStruct((1, wh3.shape[0]), jnp.float32),
  )(pool_sums, wih0, whh0, bi0, bh0, wih1, whh1, bi1, bh1,
    wh1, bhd1, wh2, bhd2, wh3, bhd3)


def kernel(snapshot_sequence, edge_index, scale, shift,
           W1, b1, W2, b2, W3, b3,
           Wih0, Whh0, bih0, bhh0, Wih1, Whh1, bih1, bhh1,
           Wh1, bh1, Wh2, bh2, Wh3, bh3):
  B, T, N, F = snapshot_sequence.shape
  D = W1.shape[0]
  G = B * T
  GN = G * N
  E = edge_index.shape[1]

  ei = edge_index.astype(jnp.int32)
  src, dst = ei[0], ei[1]

  n_ch16 = -(-E // (NS * CHA))
  n_ch16 = -(-n_ch16 // 3) * 3
  pad16 = NS * CHA * n_ch16 - E
  src_p = jnp.concatenate([src, jnp.zeros((pad16,), jnp.int32)])
  dst_p = jnp.concatenate([dst, jnp.full((pad16,), N, jnp.int32)])
  srcs_pair = jnp.stack([src_p, src_p + N]).reshape(NC, NS, n_ch16, CHA)
  dsts3 = dst_p.reshape(NS, n_ch16, CHA)

  n_ch32 = -(-E // (NC * NS * CH))
  pad32 = NC * NS * CH * n_ch32 - E
  dst_deg = jnp.concatenate(
      [dst, jnp.full((pad32,), N, jnp.int32)]).reshape(NC, NS, n_ch32, CH)

  n_pad = (N // NS + 1) * NS
  n_pad_a = N + 8
  zinit = jnp.zeros((n_pad, 16), jnp.float32)
  ones = jnp.ones((CH, 16), jnp.float32)

  p = _deg_partials(dst_deg, zinit, ones, n_pad, n_ch32)[:, :N, :]

  blk = 2000
  reps = N // blk

  xs = snapshot_sequence.reshape(GN, F)
  pools = []
  for pc in range(2):
    xp = jnp.concatenate(
        [xs[pc * N:(pc + 1) * N], xs[(pc + 2) * N:(pc + 3) * N]], axis=0)
    xw = _pre_matmul(xp, scale.reshape(1, F), shift.reshape(1, F), W1, p,
                     blk, reps)
    t = _gcn_aggregate(xw, srcs_pair, dsts3, N, n_pad_a, n_ch16, G)
    xw2 = _mid_matmul(t, b1.reshape(1, D), W2, p, blk, reps)
    t = _gcn_aggregate(xw2, srcs_pair, dsts3, N, n_pad_a, n_ch16, G)
    xw3 = _mid_matmul(t, b2.reshape(1, D), W3, p, blk, reps)
    t = _gcn_aggregate(xw3, srcs_pair, dsts3, N, n_pad_a, n_ch16, G)
    pools.append(_pool_sums(t, b3.reshape(1, D), p, blk, reps, 2))

  ps = jnp.concatenate(
      [pools[0].reshape(2, D), pools[1].reshape(2, D)])[jnp.array([0, 2, 1, 3])]
  n_out = Wh3.shape[0]
  wh3p = jnp.concatenate(
      [Wh3, jnp.zeros((128 - n_out, Wh3.shape[1]), jnp.float32)], axis=0)
  bh3p = jnp.concatenate(
      [bh3, jnp.zeros((128 - n_out,), jnp.float32)]).reshape(1, 128)
  res = _seq_head(ps, N,
                  Wih0, Whh0, bih0.reshape(1, 4 * D), bhh0.reshape(1, 4 * D),
                  Wih1, Whh1, bih1.reshape(1, 4 * D), bhh1.reshape(1, 4 * D),
                  Wh1, bh1.reshape(1, -1), Wh2, bh2.reshape(1, -1),
                  wh3p, bh3p)
  return res[:B, :n_out]
